# baseline (speedup 1.0000x reference)
import functools

import jax
import jax.numpy as jnp
from jax import lax
from jax.experimental import pallas as pl
from jax.experimental.pallas import tpu as pltpu


def _fused_kernel(ids_ref, tbl_ref, len_ref, wih_ref, bih_ref, whh_ref,
                  bhh_ref, out_ref, hid_ref, h_ref, x_ref,
                  *, TT, H, B, T_pad, U):
    d = pl.program_id(0)
    tb = pl.program_id(1)
    num_tb = pl.num_programs(1)

    wih = wih_ref[0]
    bih = bih_ref[0]
    whh = whh_ref[0]
    bhh = bhh_ref[0]
    lengths = len_ref[...]
    is_fwd = d == 0

    cur = tb % 2
    nxt = 1 - cur

    def gather_rows(buf, i, t_g):
        base = t_g * B
        for j in range(B // U):
            rows = []
            for k in range(U):
                tok = ids_ref[base + j * U + k]
                rows.append(tbl_ref[tok].reshape(1, H))
            start = pl.multiple_of(j * U, U)
            x_ref[buf, i, pl.ds(start, U), :] = jnp.concatenate(
                rows, axis=0).astype(x_ref.dtype)

    @pl.when(tb == 0)
    def _():
        h_ref[...] = jnp.zeros_like(h_ref)
        for i in range(TT):
            t_g0 = jnp.where(is_fwd, i, T_pad - 1 - i)
            gather_rows(0, i, t_g0)

    for i in range(TT):
        s = tb * TT + i
        t_g = jnp.where(is_fwd, s, T_pad - 1 - s)

        h = h_ref[...]
        x = x_ref[cur, i]
        gi = jnp.dot(x, wih, preferred_element_type=jnp.float32) + bih
        gh = jnp.dot(h.astype(jnp.bfloat16), whh,
                     preferred_element_type=jnp.float32) + bhh

        rz = jax.nn.sigmoid(gi[:, :2 * H] + gh[:, :2 * H])
        rg = rz[:, :H]
        z = rz[:, H:]
        n = jnp.tanh(gi[:, 2 * H:] + rg * gh[:, 2 * H:])
        hn = (1.0 - z) * n + z * h

        m = (t_g < lengths).astype(jnp.float32)
        o = m * hn
        r = jnp.where(is_fwd, i, TT - 1 - i)
        out_ref[0, r] = o.astype(out_ref.dtype)
        h_ref[...] = o + (1.0 - m) * h

        @pl.when(tb + 1 < num_tb)
        def _():
            s_n = (tb + 1) * TT + i
            t_n = jnp.where(is_fwd, s_n, T_pad - 1 - s_n)
            gather_rows(nxt, i, t_n)

    @pl.when(tb == num_tb - 1)
    def _():
        hid_ref[0] = h_ref[...]


def kernel(input_seq, input_lengths, embedding, wih_f, whh_f, bih_f, bhh_f,
        wih_b, whh_b, bih_b, bhh_b):
    T, B = input_seq.shape
    V, H = embedding.shape
    S = H // 128
    TT = 16
    T_pad = ((T + TT - 1) // TT) * TT
    num_tb = T_pad // TT
    N = T_pad * B
    U = min(16, B)

    ids = input_seq.reshape(T * B).astype(jnp.int32)
    if T_pad != T:
        ids = jnp.pad(ids, (0, N - T * B))
    tbl = embedding.reshape(V, S, 128)

    wih = jnp.stack([wih_f, wih_b], axis=0).astype(jnp.bfloat16)
    bih = jnp.stack([bih_f, bih_b], axis=0)
    whh = jnp.stack([whh_f, whh_b], axis=0).astype(jnp.bfloat16)
    bhh = jnp.stack([bhh_f, bhh_b], axis=0)
    lengths = input_lengths.astype(jnp.int32).reshape(B, 1)

    tbl_spec = pl.BlockSpec((V, S, 128), lambda d, t, *_: (0, 0, 0))
    len_spec = pl.BlockSpec((B, 1), lambda d, t, *_: (0, 0))
    wih_spec = pl.BlockSpec((1, H, 3 * H), lambda d, t, *_: (d, 0, 0))
    bih_spec = pl.BlockSpec((1, 1, 3 * H), lambda d, t, *_: (d, 0, 0))
    whh_spec = pl.BlockSpec((1, H, 3 * H), lambda d, t, *_: (d, 0, 0))
    bhh_spec = pl.BlockSpec((1, 1, 3 * H), lambda d, t, *_: (d, 0, 0))
    out_spec = pl.BlockSpec(
        (1, TT, B, H),
        lambda d, t, *_: (d, jnp.where(d == 0, t, num_tb - 1 - t), 0, 0))
    hid_spec = pl.BlockSpec((1, B, H), lambda d, t, *_: (d, 0, 0))

    kern = functools.partial(_fused_kernel, TT=TT, H=H, B=B, T_pad=T_pad, U=U)

    out_dir, hidden = pl.pallas_call(
        kern,
        out_shape=(
            jax.ShapeDtypeStruct((2, T_pad, B, H), jnp.bfloat16),
            jax.ShapeDtypeStruct((2, B, H), jnp.float32),
        ),
        grid_spec=pltpu.PrefetchScalarGridSpec(
            num_scalar_prefetch=1,
            grid=(2, num_tb),
            in_specs=[tbl_spec, len_spec, wih_spec, bih_spec, whh_spec,
                      bhh_spec],
            out_specs=[out_spec, hid_spec],
            scratch_shapes=[pltpu.VMEM((B, H), jnp.float32),
                            pltpu.VMEM((2, TT, B, H), jnp.bfloat16)],
        ),
        compiler_params=pltpu.CompilerParams(
            dimension_semantics=("arbitrary", "arbitrary")),
    )(ids, tbl, lengths, wih, bih, whh, bhh)

    outputs = (out_dir[0].astype(jnp.float32)
               + out_dir[1].astype(jnp.float32))[:T]
    return outputs, hidden


# speedup vs baseline: 1.2836x; 1.2836x over previous
import functools
import math

import jax
import jax.numpy as jnp
from jax import lax
from jax.experimental import pallas as pl
from jax.experimental.pallas import tpu as pltpu


def _gather_kernel(ids_ref, tbl_ref, out_ref, *, RB, U, H):
    blk = pl.program_id(0)
    base = blk * RB

    def body(j, carry):
        rows = []
        for k in range(U):
            tok = ids_ref[base + j * U + k]
            rows.append(tbl_ref[tok].reshape(1, H))
        start = pl.multiple_of(j * U, U)
        out_ref[pl.ds(start, U), :] = jnp.concatenate(
            rows, axis=0).astype(out_ref.dtype)
        return carry

    lax.fori_loop(0, RB // U, body, 0)


def _bigru_kernel(emb_ref, len_ref, wih_ref, bih_ref, whh_ref, bhh_ref,
                  out_ref, hid_ref, h_ref, *, TT, H, T_pad):
    d = pl.program_id(0)
    tb = pl.program_id(1)

    @pl.when(tb == 0)
    def _():
        h_ref[...] = jnp.zeros_like(h_ref)

    wih = wih_ref[0]
    bih = bih_ref[0]
    whh = whh_ref[0]
    bhh = bhh_ref[0]
    lengths = len_ref[...]
    is_fwd = d == 0

    for i in range(TT):
        s = tb * TT + i
        r = jnp.where(is_fwd, i, TT - 1 - i)
        t_g = jnp.where(is_fwd, s, T_pad - 1 - s)

        h = h_ref[...]
        x = emb_ref[r]
        gi = jnp.dot(x, wih, preferred_element_type=jnp.float32) + bih
        gh = jnp.dot(h.astype(jnp.bfloat16), whh,
                     preferred_element_type=jnp.float32) + bhh

        rz = jax.nn.sigmoid(gi[:, :2 * H] + gh[:, :2 * H])
        rg = rz[:, :H]
        z = rz[:, H:]
        n = jnp.tanh(gi[:, 2 * H:] + rg * gh[:, 2 * H:])
        hn = (1.0 - z) * n + z * h

        m = (t_g < lengths).astype(jnp.float32)
        o = m * hn
        out_ref[0, r] = o.astype(out_ref.dtype)
        h_ref[...] = o + (1.0 - m) * h

    @pl.when(tb == pl.num_programs(1) - 1)
    def _():
        hid_ref[0] = h_ref[...]


def kernel(input_seq, input_lengths, embedding, wih_f, whh_f, bih_f, bhh_f,
           wih_b, whh_b, bih_b, bhh_b):
    T, B = input_seq.shape
    V, H = embedding.shape
    S = H // 128
    TT = 16
    T_pad = ((T + TT - 1) // TT) * TT
    num_tb = T_pad // TT
    N = T_pad * B

    ids = input_seq.reshape(T * B).astype(jnp.int32)
    if T_pad != T:
        ids = jnp.pad(ids, (0, N - T * B))
    tbl = embedding.reshape(V, S, 128)

    RB = min(N, 4096)
    nblk = N // RB

    emb_flat = pl.pallas_call(
        functools.partial(_gather_kernel, RB=RB, U=16, H=H),
        out_shape=jax.ShapeDtypeStruct((N, H), jnp.bfloat16),
        grid_spec=pltpu.PrefetchScalarGridSpec(
            num_scalar_prefetch=1,
            grid=(nblk,),
            in_specs=[pl.BlockSpec((V, S, 128), lambda b, *_: (0, 0, 0))],
            out_specs=pl.BlockSpec((RB, H), lambda b, *_: (b, 0)),
        ),
        compiler_params=pltpu.CompilerParams(
            dimension_semantics=("arbitrary",)),
    )(ids, tbl)
    embedded = emb_flat.reshape(T_pad, B, H)

    wih = jnp.stack([wih_f, wih_b], axis=0).astype(jnp.bfloat16)
    bih = jnp.stack([bih_f, bih_b], axis=0)
    whh = jnp.stack([whh_f, whh_b], axis=0).astype(jnp.bfloat16)
    bhh = jnp.stack([bhh_f, bhh_b], axis=0)
    lengths = input_lengths.astype(jnp.int32).reshape(B, 1)

    emb_spec = pl.BlockSpec(
        (TT, B, H),
        lambda d, t: (jnp.where(d == 0, t, num_tb - 1 - t), 0, 0))
    len_spec = pl.BlockSpec((B, 1), lambda d, t: (0, 0))
    wih_spec = pl.BlockSpec((1, H, 3 * H), lambda d, t: (d, 0, 0))
    bih_spec = pl.BlockSpec((1, 1, 3 * H), lambda d, t: (d, 0, 0))
    whh_spec = pl.BlockSpec((1, H, 3 * H), lambda d, t: (d, 0, 0))
    bhh_spec = pl.BlockSpec((1, 1, 3 * H), lambda d, t: (d, 0, 0))
    out_spec = pl.BlockSpec(
        (1, TT, B, H),
        lambda d, t: (d, jnp.where(d == 0, t, num_tb - 1 - t), 0, 0))
    hid_spec = pl.BlockSpec((1, B, H), lambda d, t: (d, 0, 0))

    kern = functools.partial(_bigru_kernel, TT=TT, H=H, T_pad=T_pad)

    out_dir, hidden = pl.pallas_call(
        kern,
        out_shape=(
            jax.ShapeDtypeStruct((2, T_pad, B, H), jnp.bfloat16),
            jax.ShapeDtypeStruct((2, B, H), jnp.float32),
        ),
        grid_spec=pltpu.PrefetchScalarGridSpec(
            num_scalar_prefetch=0,
            grid=(2, num_tb),
            in_specs=[emb_spec, len_spec, wih_spec, bih_spec, whh_spec,
                      bhh_spec],
            out_specs=[out_spec, hid_spec],
            scratch_shapes=[pltpu.VMEM((B, H), jnp.float32)],
        ),
        compiler_params=pltpu.CompilerParams(
            dimension_semantics=("arbitrary", "arbitrary")),
    )(embedded, lengths, wih, bih, whh, bhh)

    outputs = (out_dir[0].astype(jnp.float32)
               + out_dir[1].astype(jnp.float32))[:T]
    return outputs, hidden


# speedup vs baseline: 1.2848x; 1.0009x over previous
import functools
import math

import jax
import jax.numpy as jnp
from jax import lax
from jax.experimental import pallas as pl
from jax.experimental.pallas import tpu as pltpu


def _gather_kernel(ids_ref, tbl_ref, out_ref, *, RB, U, H):
    blk = pl.program_id(0)
    base = blk * RB

    def body(j, carry):
        rows = []
        for k in range(U):
            tok = ids_ref[base + j * U + k]
            rows.append(tbl_ref[tok].reshape(1, H))
        start = pl.multiple_of(j * U, U)
        out_ref[pl.ds(start, U), :] = jnp.concatenate(
            rows, axis=0).astype(out_ref.dtype)
        return carry

    lax.fori_loop(0, RB // U, body, 0)


def _bigru_kernel(emb_ref, len_ref, wih_ref, bih_ref, whh_ref, bhh_ref,
                  out_ref, hid_ref, h_ref, *, TT, H, T_pad):
    d = pl.program_id(0)
    tb = pl.program_id(1)

    @pl.when(tb == 0)
    def _():
        h_ref[...] = jnp.zeros_like(h_ref)

    wih = wih_ref[0]
    bih = bih_ref[0]
    whh = whh_ref[0]
    bhh = bhh_ref[0]
    lengths = len_ref[...]
    is_fwd = d == 0

    for i in range(TT):
        s = tb * TT + i
        r = jnp.where(is_fwd, i, TT - 1 - i)
        t_g = jnp.where(is_fwd, s, T_pad - 1 - s)

        h = h_ref[...]
        x = emb_ref[r]
        gi = jnp.dot(x, wih, preferred_element_type=jnp.float32) + bih
        gh = jnp.dot(h.astype(jnp.bfloat16), whh,
                     preferred_element_type=jnp.float32) + bhh

        rz = jax.nn.sigmoid(gi[:, :2 * H] + gh[:, :2 * H])
        rg = rz[:, :H]
        z = rz[:, H:]
        n = jnp.tanh(gi[:, 2 * H:] + rg * gh[:, 2 * H:])
        hn = (1.0 - z) * n + z * h

        m = (t_g < lengths).astype(jnp.float32)
        o = m * hn
        out_ref[0, r] = o.astype(out_ref.dtype)
        h_ref[...] = o + (1.0 - m) * h

    @pl.when(tb == pl.num_programs(1) - 1)
    def _():
        hid_ref[0] = h_ref[...]


def kernel(input_seq, input_lengths, embedding, wih_f, whh_f, bih_f, bhh_f,
           wih_b, whh_b, bih_b, bhh_b):
    T, B = input_seq.shape
    V, H = embedding.shape
    S = H // 128
    TT = 16
    T_pad = ((T + TT - 1) // TT) * TT
    num_tb = T_pad // TT
    N = T_pad * B

    ids = input_seq.reshape(T * B).astype(jnp.int32)
    if T_pad != T:
        ids = jnp.pad(ids, (0, N - T * B))
    tbl = embedding.astype(jnp.bfloat16).reshape(V, S, 128)

    RB = min(N, 4096)
    nblk = N // RB

    emb_flat = pl.pallas_call(
        functools.partial(_gather_kernel, RB=RB, U=16, H=H),
        out_shape=jax.ShapeDtypeStruct((N, H), jnp.bfloat16),
        grid_spec=pltpu.PrefetchScalarGridSpec(
            num_scalar_prefetch=1,
            grid=(nblk,),
            in_specs=[pl.BlockSpec((V, S, 128), lambda b, *_: (0, 0, 0))],
            out_specs=pl.BlockSpec((RB, H), lambda b, *_: (b, 0)),
        ),
        compiler_params=pltpu.CompilerParams(
            dimension_semantics=("arbitrary",)),
    )(ids, tbl)
    embedded = emb_flat.reshape(T_pad, B, H)

    wih = jnp.stack([wih_f, wih_b], axis=0).astype(jnp.bfloat16)
    bih = jnp.stack([bih_f, bih_b], axis=0)
    whh = jnp.stack([whh_f, whh_b], axis=0).astype(jnp.bfloat16)
    bhh = jnp.stack([bhh_f, bhh_b], axis=0)
    lengths = input_lengths.astype(jnp.int32).reshape(B, 1)

    emb_spec = pl.BlockSpec(
        (TT, B, H),
        lambda d, t: (jnp.where(d == 0, t, num_tb - 1 - t), 0, 0))
    len_spec = pl.BlockSpec((B, 1), lambda d, t: (0, 0))
    wih_spec = pl.BlockSpec((1, H, 3 * H), lambda d, t: (d, 0, 0))
    bih_spec = pl.BlockSpec((1, 1, 3 * H), lambda d, t: (d, 0, 0))
    whh_spec = pl.BlockSpec((1, H, 3 * H), lambda d, t: (d, 0, 0))
    bhh_spec = pl.BlockSpec((1, 1, 3 * H), lambda d, t: (d, 0, 0))
    out_spec = pl.BlockSpec(
        (1, TT, B, H),
        lambda d, t: (d, jnp.where(d == 0, t, num_tb - 1 - t), 0, 0))
    hid_spec = pl.BlockSpec((1, B, H), lambda d, t: (d, 0, 0))

    kern = functools.partial(_bigru_kernel, TT=TT, H=H, T_pad=T_pad)

    out_dir, hidden = pl.pallas_call(
        kern,
        out_shape=(
            jax.ShapeDtypeStruct((2, T_pad, B, H), jnp.bfloat16),
            jax.ShapeDtypeStruct((2, B, H), jnp.float32),
        ),
        grid_spec=pltpu.PrefetchScalarGridSpec(
            num_scalar_prefetch=0,
            grid=(2, num_tb),
            in_specs=[emb_spec, len_spec, wih_spec, bih_spec, whh_spec,
                      bhh_spec],
            out_specs=[out_spec, hid_spec],
            scratch_shapes=[pltpu.VMEM((B, H), jnp.float32)],
        ),
        compiler_params=pltpu.CompilerParams(
            dimension_semantics=("arbitrary", "arbitrary")),
    )(embedded, lengths, wih, bih, whh, bhh)

    outputs = (out_dir[0].astype(jnp.float32)
               + out_dir[1].astype(jnp.float32))[:T]
    return outputs, hidden


# speedup vs baseline: 1.3344x; 1.0386x over previous
import functools
import math

import jax
import jax.numpy as jnp
from jax import lax
from jax.experimental import pallas as pl
from jax.experimental.pallas import tpu as pltpu


def _gather_kernel(ids_ref, tbl_ref, out_ref, *, RB, U, H):
    blk = pl.program_id(0)
    base = blk * RB

    def body(j, carry):
        rows = []
        for k in range(U):
            tok = ids_ref[base + j * U + k]
            rows.append(tbl_ref[tok].reshape(1, H))
        start = pl.multiple_of(j * U, U)
        out_ref[pl.ds(start, U), :] = jnp.concatenate(
            rows, axis=0).astype(out_ref.dtype)
        return carry

    lax.fori_loop(0, RB // U, body, 0)


def _bigru_kernel(emb_ref, len_ref, wih_f_ref, whh_f_ref, bih_f_ref,
                  bhh_f_ref, wih_b_ref, whh_b_ref, bih_b_ref, bhh_b_ref,
                  out_ref, hid_ref, h_ref, *, TT, H, T_pad):
    d = pl.program_id(0)
    tb = pl.program_id(1)

    @pl.when(tb == 0)
    def _():
        h_ref[...] = jnp.zeros_like(h_ref)

    is_fwd = d == 0
    wih = jnp.where(is_fwd, wih_f_ref[...],
                    wih_b_ref[...]).astype(jnp.bfloat16)
    whh = jnp.where(is_fwd, whh_f_ref[...],
                    whh_b_ref[...]).astype(jnp.bfloat16)
    bih = jnp.where(is_fwd, bih_f_ref[...], bih_b_ref[...])
    bhh = jnp.where(is_fwd, bhh_f_ref[...], bhh_b_ref[...])
    lengths = len_ref[...]

    for i in range(TT):
        s = tb * TT + i
        r = jnp.where(is_fwd, i, TT - 1 - i)
        t_g = jnp.where(is_fwd, s, T_pad - 1 - s)

        h = h_ref[...]
        x = emb_ref[r]
        gi = jnp.dot(x, wih, preferred_element_type=jnp.float32) + bih
        gh = jnp.dot(h.astype(jnp.bfloat16), whh,
                     preferred_element_type=jnp.float32) + bhh

        rz = jax.nn.sigmoid(gi[:, :2 * H] + gh[:, :2 * H])
        rg = rz[:, :H]
        z = rz[:, H:]
        n = jnp.tanh(gi[:, 2 * H:] + rg * gh[:, 2 * H:])
        hn = (1.0 - z) * n + z * h

        m = (t_g < lengths).astype(jnp.float32)
        o = m * hn
        out_ref[0, r] = o.astype(out_ref.dtype)
        h_ref[...] = o + (1.0 - m) * h

    @pl.when(tb == pl.num_programs(1) - 1)
    def _():
        hid_ref[0] = h_ref[...]


def kernel(input_seq, input_lengths, embedding, wih_f, whh_f, bih_f, bhh_f,
           wih_b, whh_b, bih_b, bhh_b):
    T, B = input_seq.shape
    V, H = embedding.shape
    S = H // 128
    TT = 16
    T_pad = ((T + TT - 1) // TT) * TT
    num_tb = T_pad // TT
    N = T_pad * B

    ids = input_seq.reshape(T * B).astype(jnp.int32)
    if T_pad != T:
        ids = jnp.pad(ids, (0, N - T * B))
    tbl = embedding.astype(jnp.bfloat16).reshape(V, S, 128)

    RB = min(N, 4096)
    nblk = N // RB

    emb_flat = pl.pallas_call(
        functools.partial(_gather_kernel, RB=RB, U=16, H=H),
        out_shape=jax.ShapeDtypeStruct((N, H), jnp.bfloat16),
        grid_spec=pltpu.PrefetchScalarGridSpec(
            num_scalar_prefetch=1,
            grid=(nblk,),
            in_specs=[pl.BlockSpec((V, S, 128), lambda b, *_: (0, 0, 0))],
            out_specs=pl.BlockSpec((RB, H), lambda b, *_: (b, 0)),
        ),
        compiler_params=pltpu.CompilerParams(
            dimension_semantics=("arbitrary",)),
    )(ids, tbl)
    embedded = emb_flat.reshape(T_pad, B, H)

    lengths = input_lengths.astype(jnp.int32).reshape(B, 1)

    emb_spec = pl.BlockSpec(
        (TT, B, H),
        lambda d, t: (jnp.where(d == 0, t, num_tb - 1 - t), 0, 0))
    len_spec = pl.BlockSpec((B, 1), lambda d, t: (0, 0))
    w_spec = pl.BlockSpec((H, 3 * H), lambda d, t: (0, 0))
    b_spec = pl.BlockSpec((1, 3 * H), lambda d, t: (0, 0))
    out_spec = pl.BlockSpec(
        (1, TT, B, H),
        lambda d, t: (d, jnp.where(d == 0, t, num_tb - 1 - t), 0, 0))
    hid_spec = pl.BlockSpec((1, B, H), lambda d, t: (d, 0, 0))

    kern = functools.partial(_bigru_kernel, TT=TT, H=H, T_pad=T_pad)

    out_dir, hidden = pl.pallas_call(
        kern,
        out_shape=(
            jax.ShapeDtypeStruct((2, T_pad, B, H), jnp.bfloat16),
            jax.ShapeDtypeStruct((2, B, H), jnp.float32),
        ),
        grid_spec=pltpu.PrefetchScalarGridSpec(
            num_scalar_prefetch=0,
            grid=(2, num_tb),
            in_specs=[emb_spec, len_spec, w_spec, w_spec, b_spec, b_spec,
                      w_spec, w_spec, b_spec, b_spec],
            out_specs=[out_spec, hid_spec],
            scratch_shapes=[pltpu.VMEM((B, H), jnp.float32)],
        ),
        compiler_params=pltpu.CompilerParams(
            dimension_semantics=("arbitrary", "arbitrary")),
    )(embedded, lengths, wih_f, whh_f, bih_f, bhh_f,
      wih_b, whh_b, bih_b, bhh_b)

    outputs = (out_dir[0].astype(jnp.float32)
               + out_dir[1].astype(jnp.float32))[:T]
    return outputs, hidden


# speedup vs baseline: 1.3371x; 1.0020x over previous
import functools
import math

import jax
import jax.numpy as jnp
from jax import lax
from jax.experimental import pallas as pl
from jax.experimental.pallas import tpu as pltpu


def _gather_kernel(ids_ref, tbl_ref, out_ref, *, RB, U, H):
    blk = pl.program_id(0)
    base = blk * RB

    def body(j, carry):
        rows = []
        for k in range(U):
            tok = ids_ref[base + j * U + k]
            rows.append(tbl_ref[tok].reshape(1, H))
        start = pl.multiple_of(j * U, U)
        out_ref[pl.ds(start, U), :] = jnp.concatenate(
            rows, axis=0).astype(out_ref.dtype)
        return carry

    lax.fori_loop(0, RB // U, body, 0)


def _bigru_kernel(emb_ref, len_ref, wih_f_ref, whh_f_ref, bih_f_ref,
                  bhh_f_ref, wih_b_ref, whh_b_ref, bih_b_ref, bhh_b_ref,
                  out_ref, hid_ref, h_ref, *, TT, H, T_pad):
    d = pl.program_id(0)
    tb = pl.program_id(1)

    @pl.when(tb == 0)
    def _():
        h_ref[...] = jnp.zeros_like(h_ref)

    is_fwd = d == 0
    wih = jnp.where(is_fwd, wih_f_ref[...],
                    wih_b_ref[...]).astype(jnp.bfloat16)
    whh = jnp.where(is_fwd, whh_f_ref[...],
                    whh_b_ref[...]).astype(jnp.bfloat16)
    bih = jnp.where(is_fwd, bih_f_ref[...], bih_b_ref[...])
    bhh = jnp.where(is_fwd, bhh_f_ref[...], bhh_b_ref[...])
    lengths = len_ref[...]

    for i in range(TT):
        s = tb * TT + i
        r = jnp.where(is_fwd, i, TT - 1 - i)
        t_g = jnp.where(is_fwd, s, T_pad - 1 - s)

        h = h_ref[...]
        x = emb_ref[r]
        gi = jnp.dot(x, wih, preferred_element_type=jnp.float32) + bih
        gh = jnp.dot(h.astype(jnp.bfloat16), whh,
                     preferred_element_type=jnp.float32) + bhh

        rz = jax.nn.sigmoid(gi[:, :2 * H] + gh[:, :2 * H])
        rg = rz[:, :H]
        z = rz[:, H:]
        n = jnp.tanh(gi[:, 2 * H:] + rg * gh[:, 2 * H:])
        hn = (1.0 - z) * n + z * h

        m = (t_g < lengths).astype(jnp.float32)
        o = m * hn
        out_ref[0, r] = o.astype(out_ref.dtype)
        h_ref[...] = o + (1.0 - m) * h

    @pl.when(tb == pl.num_programs(1) - 1)
    def _():
        hid_ref[0] = h_ref[...]


def kernel(input_seq, input_lengths, embedding, wih_f, whh_f, bih_f, bhh_f,
           wih_b, whh_b, bih_b, bhh_b):
    T, B = input_seq.shape
    V, H = embedding.shape
    S = H // 128
    TT = 32
    T_pad = ((T + TT - 1) // TT) * TT
    num_tb = T_pad // TT
    N = T_pad * B

    ids = input_seq.reshape(T * B).astype(jnp.int32)
    if T_pad != T:
        ids = jnp.pad(ids, (0, N - T * B))
    tbl = embedding.astype(jnp.bfloat16).reshape(V, S, 128)

    RB = min(N, 4096)
    nblk = N // RB

    emb_flat = pl.pallas_call(
        functools.partial(_gather_kernel, RB=RB, U=16, H=H),
        out_shape=jax.ShapeDtypeStruct((N, H), jnp.bfloat16),
        grid_spec=pltpu.PrefetchScalarGridSpec(
            num_scalar_prefetch=1,
            grid=(nblk,),
            in_specs=[pl.BlockSpec((V, S, 128), lambda b, *_: (0, 0, 0))],
            out_specs=pl.BlockSpec((RB, H), lambda b, *_: (b, 0)),
        ),
        compiler_params=pltpu.CompilerParams(
            dimension_semantics=("arbitrary",)),
    )(ids, tbl)
    embedded = emb_flat.reshape(T_pad, B, H)

    lengths = input_lengths.astype(jnp.int32).reshape(B, 1)

    emb_spec = pl.BlockSpec(
        (TT, B, H),
        lambda d, t: (jnp.where(d == 0, t, num_tb - 1 - t), 0, 0))
    len_spec = pl.BlockSpec((B, 1), lambda d, t: (0, 0))
    w_spec = pl.BlockSpec((H, 3 * H), lambda d, t: (0, 0))
    b_spec = pl.BlockSpec((1, 3 * H), lambda d, t: (0, 0))
    out_spec = pl.BlockSpec(
        (1, TT, B, H),
        lambda d, t: (d, jnp.where(d == 0, t, num_tb - 1 - t), 0, 0))
    hid_spec = pl.BlockSpec((1, B, H), lambda d, t: (d, 0, 0))

    kern = functools.partial(_bigru_kernel, TT=TT, H=H, T_pad=T_pad)

    out_dir, hidden = pl.pallas_call(
        kern,
        out_shape=(
            jax.ShapeDtypeStruct((2, T_pad, B, H), jnp.bfloat16),
            jax.ShapeDtypeStruct((2, B, H), jnp.float32),
        ),
        grid_spec=pltpu.PrefetchScalarGridSpec(
            num_scalar_prefetch=0,
            grid=(2, num_tb),
            in_specs=[emb_spec, len_spec, w_spec, w_spec, b_spec, b_spec,
                      w_spec, w_spec, b_spec, b_spec],
            out_specs=[out_spec, hid_spec],
            scratch_shapes=[pltpu.VMEM((B, H), jnp.float32)],
        ),
        compiler_params=pltpu.CompilerParams(
            dimension_semantics=("arbitrary", "arbitrary")),
    )(embedded, lengths, wih_f, whh_f, bih_f, bhh_f,
      wih_b, whh_b, bih_b, bhh_b)

    outputs = (out_dir[0].astype(jnp.float32)
               + out_dir[1].astype(jnp.float32))[:T]
    return outputs, hidden
